# baseline (speedup 1.0000x reference)
import jax
import jax.numpy as jnp
from jax import lax
from jax.experimental import pallas as pl
from jax.experimental.pallas import tpu as pltpu
from jax.experimental.pallas import tpu_sc as plsc

BS = 2
N = 10000
D = 128
NC = 2
NS = 16
CE = 80
KR = 3
KE = 4
SLOTS = 12

NODE_T = ((N + NS * 16 - 1) // (NS * 16)) * 16
NPAD = NODE_T * NS


def _sc_body(src_hbm, dst_hbm, ew_hbm, x_hbm,
             agg_o, cn_o, dinv_o,
             agg_sh, deg_sh, dinv_sh, cn_sh,
             dinv_t, dbuf,
             srcd0, srcd1, srcd2, srcd3,
             dstd0, dstd1, dstd2, dstd3,
             ewd0, ewd1, ewd2, ewd3,
             srcb0, srcb1, srcb2, srcb3,
             normd0, normd1, normd2, normd3,
             rows0, rows1, rows2,
             gsem0, gsem1, gsem2, ssem0, ssem1, ssem2,
             esem0, esem1, esem2, esem3, csem0, csem1, csem2, csem3,
             asem):
    srcd = [srcd0, srcd1, srcd2, srcd3]
    dstd = [dstd0, dstd1, dstd2, dstd3]
    ewd = [ewd0, ewd1, ewd2, ewd3]
    srcb = [srcb0, srcb1, srcb2, srcb3]
    normd = [normd0, normd1, normd2, normd3]
    rows = [rows0, rows1, rows2]
    gsem = [gsem0, gsem1, gsem2]
    ssem = [ssem0, ssem1, ssem2]
    esem = [esem0, esem1, esem2, esem3]
    csem = [csem0, csem1, csem2, csem3]

    ept = src_hbm.shape[0] // NS
    m = ept // CE
    cid = lax.axis_index("c")
    sid = lax.axis_index("s")
    tbase = sid * NODE_T
    boff = cid * N

    z16 = jnp.zeros((16,), jnp.float32)

    @pl.loop(0, CE)
    def _(r):
        for u in range(D // 16):
            rows0[r, pl.ds(u * 16, 16)] = z16

    for i in range(128 // 16):
        dbuf[pl.ds(i * 16, 16)] = z16

    @pl.loop(0, NODE_T // CE)
    def _(i):
        pltpu.async_copy(rows0, agg_sh.at[pl.ds(tbase + i * CE, CE)], asem)

    @pl.loop(0, NODE_T // 128)
    def _(i):
        pltpu.async_copy(dbuf, deg_sh.at[pl.ds(tbase + i * 128, 128)], asem)
        pltpu.async_copy(dbuf, cn_sh.at[pl.ds(tbase + i * 128, 128)], asem)

    @pl.loop(0, NODE_T // CE)
    def _(i):
        pltpu.make_async_copy(
            rows0, agg_sh.at[pl.ds(tbase + i * CE, CE)], asem).wait()

    @pl.loop(0, NODE_T // 128)
    def _(i):
        pltpu.make_async_copy(
            dbuf, deg_sh.at[pl.ds(tbase + i * 128, 128)], asem).wait()
        pltpu.make_async_copy(
            dbuf, cn_sh.at[pl.ds(tbase + i * 128, 128)], asem).wait()

    plsc.subcore_barrier()

    def prefetchA(c, b):
        @pl.when(c >= KE)
        def _():
            pltpu.make_async_copy(ewd[b], deg_sh.at[dstd[b]], asem).wait()

        base = sid * ept + c * CE
        pltpu.async_copy(dst_hbm.at[pl.ds(base, CE)], dstd[b], esem[b])
        pltpu.async_copy(ew_hbm.at[pl.ds(base, CE)], ewd[b], esem[b])

    def stepA(c, b):
        base = sid * ept + c * CE
        pltpu.make_async_copy(dst_hbm.at[pl.ds(base, CE)], dstd[b],
                              esem[b]).wait()
        pltpu.make_async_copy(ew_hbm.at[pl.ds(base, CE)], ewd[b],
                              esem[b]).wait()
        pltpu.async_copy(ewd[b], deg_sh.at[dstd[b]], asem, add=True)

    prefetchA(0, 0)
    prefetchA(1, 1)

    @pl.loop(0, m // KE)
    def _(t4):
        c0 = t4 * KE
        for p in range(KE):
            c = c0 + p
            stepA(c, p)
            cp = c + 2

            @pl.when(cp < m)
            def _():
                prefetchA(cp, (p + 2) % KE)

    for c in range(KE):
        cc = m - KE + c
        pltpu.make_async_copy(ewd[cc % KE], deg_sh.at[dstd[cc % KE]],
                              asem).wait()

    plsc.subcore_barrier()

    @pl.loop(0, NODE_T // 128)
    def _(half):
        hb = tbase + half * 128
        pltpu.sync_copy(deg_sh.at[pl.ds(hb, 128)], dbuf)

        @pl.loop(0, 128 // 16)
        def _(i):
            v = dbuf[pl.ds(i * 16, 16)] + 1.0
            iv = lax.bitcast_convert_type(v, jnp.int32)
            iv = 0x5F3759DF - lax.shift_right_logical(iv, 1)
            y = lax.bitcast_convert_type(iv, jnp.float32)
            for _ in range(3):
                y = y * (1.5 - 0.5 * v * y * y)
            dbuf[pl.ds(i * 16, 16)] = y

        pltpu.sync_copy(dbuf, dinv_sh.at[pl.ds(hb, 128)])

        @pl.when(cid == 0)
        def _():
            pltpu.sync_copy(dbuf, dinv_o.at[pl.ds(hb, 128)])

    plsc.subcore_barrier()
    pltpu.sync_copy(dinv_sh.at[pl.ds(0, dinv_t.shape[0])], dinv_t)

    def prefetch(c, b):
        base = sid * ept + c * CE
        pltpu.async_copy(src_hbm.at[pl.ds(base, CE)], srcd[b], esem[b])
        pltpu.async_copy(dst_hbm.at[pl.ds(base, CE)], dstd[b], esem[b])
        pltpu.async_copy(ew_hbm.at[pl.ds(base, CE)], ewd[b], esem[b])

    def launch(c, b, rb):
        base = sid * ept + c * CE
        pltpu.make_async_copy(src_hbm.at[pl.ds(base, CE)], srcd[b],
                              esem[b]).wait()
        pltpu.make_async_copy(dst_hbm.at[pl.ds(base, CE)], dstd[b],
                              esem[b]).wait()
        pltpu.make_async_copy(ew_hbm.at[pl.ds(base, CE)], ewd[b],
                              esem[b]).wait()

        @pl.loop(0, CE // 16)
        def _(gg):
            srcv = srcd[b][pl.ds(gg * 16, 16)]
            dstv = dstd[b][pl.ds(gg * 16, 16)]
            eww = ewd[b][pl.ds(gg * 16, 16)]
            nv = plsc.load_gather(dinv_t, [srcv]) * eww * \
                plsc.load_gather(dinv_t, [dstv])
            normd[b][pl.ds(gg * 16, 16)] = nv
            srcb[b][pl.ds(gg * 16, 16)] = srcv + boff

        pltpu.async_copy(normd[b], cn_sh.at[srcd[b]], csem[b], add=True)
        pltpu.async_copy(x_hbm.at[srcb[b]], rows[rb], gsem[rb])

    def finish(c, b, rb):
        pltpu.make_async_copy(x_hbm.at[srcb[b]], rows[rb], gsem[rb]).wait()

        @pl.loop(0, CE // 16)
        def _(g):
            normv = normd[b][pl.ds(g * 16, 16)]
            for r in range(16):
                nb = jnp.take_along_axis(
                    normv, jnp.full((16, 1), r, jnp.int32)[:, 0], axis=0,
                    mode="promise_in_bounds")
                for u in range(D // 16):
                    rows[rb][g * 16 + r, pl.ds(u * 16, 16)] = \
                        rows[rb][g * 16 + r, pl.ds(u * 16, 16)] * nb

        pltpu.async_copy(rows[rb], agg_sh.at[dstd[b]], ssem[rb], add=True)

    def drain_scat(c, b, rb):
        pltpu.make_async_copy(rows[rb], agg_sh.at[dstd[b]], ssem[rb]).wait()

    def drain_cn(c, b):
        pltpu.make_async_copy(normd[b], cn_sh.at[srcd[b]], csem[b]).wait()

    prefetch(0, 0)
    prefetch(1, 1)
    launch(0, 0, 0)

    @pl.loop(0, m // SLOTS)
    def _(t):
        y0 = t * SLOTS
        for jj in range(SLOTS):
            y = y0 + jj
            yl = y + 1
            bl, rbl = (jj + 1) % KE, (jj + 1) % KR

            @pl.when(yl < m)
            def _():
                @pl.when(yl >= KR)
                def _():
                    drain_scat(yl - KR, (jj + 1 - KR) % KE,
                               (jj + 1 - KR) % KR)

                @pl.when(yl >= KR)
                def _():
                    drain_cn(yl - KR, (jj + 1 - KR) % KE)

                launch(yl, bl, rbl)

            finish(y, jj % KE, jj % KR)

            yp = y + 2

            @pl.when(yp < m)
            def _():
                prefetch(yp, (jj + 2) % KE)

    for c in range(KR):
        cc = m - KR + c
        drain_scat(cc, cc % KE, cc % KR)
    for c in range(KR):
        cc = m - KR + c
        drain_cn(cc, cc % KE)

    plsc.subcore_barrier()

    pltpu.sync_copy(agg_sh.at[pl.ds(tbase, NODE_T)],
                    agg_o.at[pl.ds(cid * NPAD + tbase, NODE_T)])

    @pl.when(cid == 0)
    def _():
        pltpu.sync_copy(cn_sh.at[pl.ds(tbase, NODE_T)],
                        cn_o.at[pl.ds(tbase, NODE_T)])


def _make_sc_kernel(interpret=False):
    dma = pltpu.SemaphoreType.DMA
    i32 = jnp.int32
    f32 = jnp.float32
    return pl.kernel(
        _sc_body,
        out_type=(
            jax.ShapeDtypeStruct((BS * NPAD, D), f32),
            jax.ShapeDtypeStruct((NPAD,), f32),
            jax.ShapeDtypeStruct((NPAD,), f32),
        ),
        mesh=plsc.VectorSubcoreMesh(core_axis_name="c", subcore_axis_name="s",
                                    num_cores=NC),
        scratch_types=(
            [
                pltpu.VMEM_SHARED((NPAD, D), f32),
                pltpu.VMEM_SHARED((NPAD,), f32),
                pltpu.VMEM_SHARED((NPAD,), f32),
                pltpu.VMEM_SHARED((NPAD,), f32),
                pltpu.VMEM((N,), f32),
                pltpu.VMEM((128,), f32),
            ]
            + [pltpu.VMEM((CE,), i32) for _ in range(KE)]
            + [pltpu.VMEM((CE,), i32) for _ in range(KE)]
            + [pltpu.VMEM((CE,), f32) for _ in range(KE)]
            + [pltpu.VMEM((CE,), i32) for _ in range(KE)]
            + [pltpu.VMEM((CE,), f32) for _ in range(KE)]
            + [pltpu.VMEM((CE, D), f32) for _ in range(KR)]
            + [dma] * (2 * KR + 2 * KE + 1)
        ),
        compiler_params=pltpu.CompilerParams(needs_layout_passes=False),
        interpret=interpret,
    )


def _tc_body(x_ref, agg_ref, cn_ref, dinv_ref, w1_ref, b1_ref, w2_ref, b2_ref,
             o_ref):
    dv = dinv_ref[...]
    sl = dv * dv
    c = cn_ref[...] + sl
    w1 = w1_ref[...]
    w2 = w2_ref[...]
    for b in range(BS):
        t = agg_ref[b, :N, :] + sl * x_ref[b]
        h = jnp.maximum(
            jnp.dot(t, w1, preferred_element_type=jnp.float32) + b1_ref[...],
            0.0)
        p = jnp.sum(c * h, axis=0, keepdims=True)
        o_ref[b:b + 1, :] = (
            jnp.dot(p, w2, preferred_element_type=jnp.float32) * (1.0 / N)
            + b2_ref[...])


def _tc_call(x, agg, cn2, dinv2, W1, b1r, W2, b2r, interpret=False):
    return pl.pallas_call(
        _tc_body,
        out_shape=jax.ShapeDtypeStruct((BS, D), jnp.float32),
        interpret=interpret,
    )(x, agg, cn2, dinv2, W1, b1r, W2, b2r)


def _run(node_features, edge_index, edge_features, W1, b1, W2, b2,
         sc_interpret=False, tc_interpret=False):
    e = edge_index.shape[1]
    ew = edge_features[:, -1]

    m = ((e + NS * CE - 1) // (NS * CE) + SLOTS - 1) // SLOTS * SLOTS
    e_padC = NS * m * CE
    src1 = jnp.pad(edge_index[0], (0, e_padC - e))
    dst1 = jnp.pad(edge_index[1], (0, e_padC - e))
    ew1 = jnp.pad(ew, (0, e_padC - e))

    xflat = node_features.reshape(BS * N, D)

    sc = _make_sc_kernel(interpret=sc_interpret)
    agg_flat, cn, dinv = sc(src1, dst1, ew1, xflat)
    agg = agg_flat.reshape(BS, NPAD, D)

    cn2 = cn[:N, None]
    dinv2 = dinv[:N, None]
    return _tc_call(node_features, agg, cn2, dinv2, W1, b1[None, :], W2,
                    b2[None, :], interpret=tc_interpret)


def kernel(node_features, edge_index, edge_features, W1, b1, W2, b2):
    return _run(node_features, edge_index, edge_features, W1, b1, W2, b2)


# speedup vs baseline: 1.0471x; 1.0471x over previous
import jax
import jax.numpy as jnp
from jax import lax
from jax.experimental import pallas as pl
from jax.experimental.pallas import tpu as pltpu
from jax.experimental.pallas import tpu_sc as plsc

BS = 2
N = 10000
D = 128
NC = 2
NS = 16
CE = 80
KR = 3
KE = 4
SLOTS = 12
CEA = 128
GRPA = 8

NODE_T = ((N + NS * 16 - 1) // (NS * 16)) * 16
NPAD = NODE_T * NS


def _sc_body(dstA_hbm, ewA_hbm, src_hbm, dst_hbm, ew_hbm, x_hbm,
             agg_o, cn_o, dinv_o,
             agg_sh, deg_sh, dinv_sh, cn_sh,
             dst8, ew8, dinv_t, dbuf,
             srcd0, srcd1, srcd2, srcd3,
             dstd0, dstd1, dstd2, dstd3,
             ewd0, ewd1, ewd2, ewd3,
             srcb0, srcb1, srcb2, srcb3,
             normd0, normd1, normd2, normd3,
             rows0, rows1, rows2,
             gsem0, gsem1, gsem2, ssem0, ssem1, ssem2,
             esem0, esem1, esem2, esem3, csem0, csem1, csem2, csem3,
             asem):
    srcd = [srcd0, srcd1, srcd2, srcd3]
    dstd = [dstd0, dstd1, dstd2, dstd3]
    ewd = [ewd0, ewd1, ewd2, ewd3]
    srcb = [srcb0, srcb1, srcb2, srcb3]
    normd = [normd0, normd1, normd2, normd3]
    rows = [rows0, rows1, rows2]
    gsem = [gsem0, gsem1, gsem2]
    ssem = [ssem0, ssem1, ssem2]
    esem = [esem0, esem1, esem2, esem3]
    csem = [csem0, csem1, csem2, csem3]

    nchA = dstA_hbm.shape[0] // NS
    ngA = nchA // GRPA
    ept = src_hbm.shape[0] // NS
    m = ept // CE
    cid = lax.axis_index("c")
    sid = lax.axis_index("s")
    tbase = sid * NODE_T
    boff = cid * N

    z16 = jnp.zeros((16,), jnp.float32)

    @pl.loop(0, CE)
    def _(r):
        for u in range(D // 16):
            rows0[r, pl.ds(u * 16, 16)] = z16

    for i in range(128 // 16):
        dbuf[pl.ds(i * 16, 16)] = z16

    @pl.loop(0, NODE_T // CE)
    def _(i):
        pltpu.async_copy(rows0, agg_sh.at[pl.ds(tbase + i * CE, CE)], asem)

    @pl.loop(0, NODE_T // 128)
    def _(i):
        pltpu.async_copy(dbuf, deg_sh.at[pl.ds(tbase + i * 128, 128)], asem)
        pltpu.async_copy(dbuf, cn_sh.at[pl.ds(tbase + i * 128, 128)], asem)

    @pl.loop(0, NODE_T // CE)
    def _(i):
        pltpu.make_async_copy(
            rows0, agg_sh.at[pl.ds(tbase + i * CE, CE)], asem).wait()

    @pl.loop(0, NODE_T // 128)
    def _(i):
        pltpu.make_async_copy(
            dbuf, deg_sh.at[pl.ds(tbase + i * 128, 128)], asem).wait()
        pltpu.make_async_copy(
            dbuf, cn_sh.at[pl.ds(tbase + i * 128, 128)], asem).wait()

    plsc.subcore_barrier()

    @pl.loop(0, ngA)
    def _(g):
        base = sid * nchA + g * GRPA
        pltpu.sync_copy(dstA_hbm.at[pl.ds(base, GRPA)], dst8)
        pltpu.sync_copy(ewA_hbm.at[pl.ds(base, GRPA)], ew8)

        @pl.loop(0, GRPA)
        def _(k):
            pltpu.async_copy(ew8.at[k], deg_sh.at[dst8.at[k]], asem,
                             add=True)

        @pl.loop(0, GRPA)
        def _(k):
            pltpu.make_async_copy(ew8.at[k], deg_sh.at[dst8.at[k]],
                                  asem).wait()

    plsc.subcore_barrier()

    @pl.loop(0, NODE_T // 128)
    def _(half):
        hb = tbase + half * 128
        pltpu.sync_copy(deg_sh.at[pl.ds(hb, 128)], dbuf)

        @pl.loop(0, 128 // 16)
        def _(i):
            v = dbuf[pl.ds(i * 16, 16)] + 1.0
            iv = lax.bitcast_convert_type(v, jnp.int32)
            iv = 0x5F3759DF - lax.shift_right_logical(iv, 1)
            y = lax.bitcast_convert_type(iv, jnp.float32)
            for _ in range(3):
                y = y * (1.5 - 0.5 * v * y * y)
            dbuf[pl.ds(i * 16, 16)] = y

        pltpu.sync_copy(dbuf, dinv_sh.at[pl.ds(hb, 128)])

        @pl.when(cid == 0)
        def _():
            pltpu.sync_copy(dbuf, dinv_o.at[pl.ds(hb, 128)])

    plsc.subcore_barrier()
    pltpu.sync_copy(dinv_sh.at[pl.ds(0, dinv_t.shape[0])], dinv_t)

    def prefetch(c, b):
        base = sid * ept + c * CE
        pltpu.async_copy(src_hbm.at[pl.ds(base, CE)], srcd[b], esem[b])
        pltpu.async_copy(dst_hbm.at[pl.ds(base, CE)], dstd[b], esem[b])
        pltpu.async_copy(ew_hbm.at[pl.ds(base, CE)], ewd[b], esem[b])

    def launch(c, b, rb):
        base = sid * ept + c * CE
        pltpu.make_async_copy(src_hbm.at[pl.ds(base, CE)], srcd[b],
                              esem[b]).wait()
        pltpu.make_async_copy(dst_hbm.at[pl.ds(base, CE)], dstd[b],
                              esem[b]).wait()
        pltpu.make_async_copy(ew_hbm.at[pl.ds(base, CE)], ewd[b],
                              esem[b]).wait()

        @pl.loop(0, CE // 16)
        def _(gg):
            srcv = srcd[b][pl.ds(gg * 16, 16)]
            dstv = dstd[b][pl.ds(gg * 16, 16)]
            eww = ewd[b][pl.ds(gg * 16, 16)]
            nv = plsc.load_gather(dinv_t, [srcv]) * eww * \
                plsc.load_gather(dinv_t, [dstv])
            normd[b][pl.ds(gg * 16, 16)] = nv
            srcb[b][pl.ds(gg * 16, 16)] = srcv + boff

        pltpu.async_copy(normd[b], cn_sh.at[srcd[b]], csem[b], add=True)
        pltpu.async_copy(x_hbm.at[srcb[b]], rows[rb], gsem[rb])

    def finish(c, b, rb):
        pltpu.make_async_copy(x_hbm.at[srcb[b]], rows[rb], gsem[rb]).wait()

        @pl.loop(0, CE // 16)
        def _(g):
            normv = normd[b][pl.ds(g * 16, 16)]
            for r in range(16):
                nb = jnp.take_along_axis(
                    normv, jnp.full((16, 1), r, jnp.int32)[:, 0], axis=0,
                    mode="promise_in_bounds")
                for u in range(D // 16):
                    rows[rb][g * 16 + r, pl.ds(u * 16, 16)] = \
                        rows[rb][g * 16 + r, pl.ds(u * 16, 16)] * nb

        pltpu.async_copy(rows[rb], agg_sh.at[dstd[b]], ssem[rb], add=True)

    def drain_scat(c, b, rb):
        pltpu.make_async_copy(rows[rb], agg_sh.at[dstd[b]], ssem[rb]).wait()

    def drain_cn(c, b):
        pltpu.make_async_copy(normd[b], cn_sh.at[srcd[b]], csem[b]).wait()

    prefetch(0, 0)
    prefetch(1, 1)
    launch(0, 0, 0)

    @pl.loop(0, m // SLOTS)
    def _(t):
        y0 = t * SLOTS
        for jj in range(SLOTS):
            y = y0 + jj
            yl = y + 1
            bl, rbl = (jj + 1) % KE, (jj + 1) % KR

            @pl.when(yl < m)
            def _():
                @pl.when(yl >= KR)
                def _():
                    drain_scat(yl - KR, (jj + 1 - KR) % KE,
                               (jj + 1 - KR) % KR)

                @pl.when(yl >= KR)
                def _():
                    drain_cn(yl - KR, (jj + 1 - KR) % KE)

                launch(yl, bl, rbl)

            finish(y, jj % KE, jj % KR)

            yp = y + 2

            @pl.when(yp < m)
            def _():
                prefetch(yp, (jj + 2) % KE)

    for c in range(KR):
        cc = m - KR + c
        drain_scat(cc, cc % KE, cc % KR)
    for c in range(KR):
        cc = m - KR + c
        drain_cn(cc, cc % KE)

    plsc.subcore_barrier()

    pltpu.sync_copy(agg_sh.at[pl.ds(tbase, NODE_T)],
                    agg_o.at[pl.ds(cid * NPAD + tbase, NODE_T)])

    @pl.when(cid == 0)
    def _():
        pltpu.sync_copy(cn_sh.at[pl.ds(tbase, NODE_T)],
                        cn_o.at[pl.ds(tbase, NODE_T)])


def _make_sc_kernel(interpret=False):
    dma = pltpu.SemaphoreType.DMA
    i32 = jnp.int32
    f32 = jnp.float32
    return pl.kernel(
        _sc_body,
        out_type=(
            jax.ShapeDtypeStruct((BS * NPAD, D), f32),
            jax.ShapeDtypeStruct((NPAD,), f32),
            jax.ShapeDtypeStruct((NPAD,), f32),
        ),
        mesh=plsc.VectorSubcoreMesh(core_axis_name="c", subcore_axis_name="s",
                                    num_cores=NC),
        scratch_types=(
            [
                pltpu.VMEM_SHARED((NPAD, D), f32),
                pltpu.VMEM_SHARED((NPAD,), f32),
                pltpu.VMEM_SHARED((NPAD,), f32),
                pltpu.VMEM_SHARED((NPAD,), f32),
                pltpu.VMEM((GRPA, CEA), i32),
                pltpu.VMEM((GRPA, CEA), f32),
                pltpu.VMEM((N,), f32),
                pltpu.VMEM((128,), f32),
            ]
            + [pltpu.VMEM((CE,), i32) for _ in range(KE)]
            + [pltpu.VMEM((CE,), i32) for _ in range(KE)]
            + [pltpu.VMEM((CE,), f32) for _ in range(KE)]
            + [pltpu.VMEM((CE,), i32) for _ in range(KE)]
            + [pltpu.VMEM((CE,), f32) for _ in range(KE)]
            + [pltpu.VMEM((CE, D), f32) for _ in range(KR)]
            + [dma] * (2 * KR + 2 * KE + 1)
        ),
        compiler_params=pltpu.CompilerParams(needs_layout_passes=False),
        interpret=interpret,
    )


def _tc_body(x_ref, agg_ref, cn_ref, dinv_ref, w1_ref, b1_ref, w2_ref, b2_ref,
             o_ref):
    dv = dinv_ref[...]
    sl = dv * dv
    c = cn_ref[...] + sl
    w1 = w1_ref[...]
    w2 = w2_ref[...]
    for b in range(BS):
        t = agg_ref[b, :N, :] + sl * x_ref[b]
        h = jnp.maximum(
            jnp.dot(t, w1, preferred_element_type=jnp.float32) + b1_ref[...],
            0.0)
        p = jnp.sum(c * h, axis=0, keepdims=True)
        o_ref[b:b + 1, :] = (
            jnp.dot(p, w2, preferred_element_type=jnp.float32) * (1.0 / N)
            + b2_ref[...])


def _tc_call(x, agg, cn2, dinv2, W1, b1r, W2, b2r, interpret=False):
    return pl.pallas_call(
        _tc_body,
        out_shape=jax.ShapeDtypeStruct((BS, D), jnp.float32),
        interpret=interpret,
    )(x, agg, cn2, dinv2, W1, b1r, W2, b2r)


def _run(node_features, edge_index, edge_features, W1, b1, W2, b2,
         sc_interpret=False, tc_interpret=False):
    e = edge_index.shape[1]
    ew = edge_features[:, -1]

    nchA = (((e + NS * CEA - 1) // (NS * CEA)) + GRPA - 1) // GRPA * GRPA
    e_padA = NS * nchA * CEA
    dstA = jnp.pad(edge_index[1], (0, e_padA - e)).reshape(NS * nchA, CEA)
    ewA = jnp.pad(ew, (0, e_padA - e)).reshape(NS * nchA, CEA)

    m = ((e + NS * CE - 1) // (NS * CE) + SLOTS - 1) // SLOTS * SLOTS
    e_padC = NS * m * CE
    src1 = jnp.pad(edge_index[0], (0, e_padC - e))
    dst1 = jnp.pad(edge_index[1], (0, e_padC - e))
    ew1 = jnp.pad(ew, (0, e_padC - e))

    xflat = node_features.reshape(BS * N, D)

    sc = _make_sc_kernel(interpret=sc_interpret)
    agg_flat, cn, dinv = sc(dstA, ewA, src1, dst1, ew1, xflat)
    agg = agg_flat.reshape(BS, NPAD, D)

    cn2 = cn[:N, None]
    dinv2 = dinv[:N, None]
    return _tc_call(node_features, agg, cn2, dinv2, W1, b1[None, :], W2,
                    b2[None, :], interpret=tc_interpret)


def kernel(node_features, edge_index, edge_features, W1, b1, W2, b2):
    return _run(node_features, edge_index, edge_features, W1, b1, W2, b2)


# speedup vs baseline: 1.0710x; 1.0228x over previous
import jax
import jax.numpy as jnp
from jax import lax
from jax.experimental import pallas as pl
from jax.experimental.pallas import tpu as pltpu
from jax.experimental.pallas import tpu_sc as plsc

BS = 2
N = 10000
D = 128
NC = 2
NS = 16
CE = 80
KR = 3
KE = 6
SLOTS = 6
CEA = 128
GRPA = 8

NODE_T = ((N + NS * 16 - 1) // (NS * 16)) * 16
NPAD = NODE_T * NS


def _sc_body(dstA_hbm, ewA_hbm, src_hbm, dst_hbm, ew_hbm, x_hbm,
             agg_o, cn_o, dinv_o,
             agg_sh, deg_sh, dinv_sh, cn_sh,
             dst8, ew8, dinv_t, dbuf,
             srcd0, srcd1, srcd2, srcd3, srcd4, srcd5,
             dstd0, dstd1, dstd2, dstd3, dstd4, dstd5,
             ewd0, ewd1, ewd2, ewd3, ewd4, ewd5,
             srcb0, srcb1, srcb2, srcb3, srcb4, srcb5,
             normd0, normd1, normd2, normd3, normd4, normd5,
             rows0, rows1, rows2,
             gsem0, gsem1, gsem2, ssem0, ssem1, ssem2,
             esem0, esem1, esem2, esem3, esem4, esem5,
             csem0, csem1, csem2, csem3, csem4, csem5,
             asem):
    srcd = [srcd0, srcd1, srcd2, srcd3, srcd4, srcd5]
    dstd = [dstd0, dstd1, dstd2, dstd3, dstd4, dstd5]
    ewd = [ewd0, ewd1, ewd2, ewd3, ewd4, ewd5]
    srcb = [srcb0, srcb1, srcb2, srcb3, srcb4, srcb5]
    normd = [normd0, normd1, normd2, normd3, normd4, normd5]
    rows = [rows0, rows1, rows2]
    gsem = [gsem0, gsem1, gsem2]
    ssem = [ssem0, ssem1, ssem2]
    esem = [esem0, esem1, esem2, esem3, esem4, esem5]
    csem = [csem0, csem1, csem2, csem3, csem4, csem5]

    nchA = dstA_hbm.shape[0] // NS
    ngA = nchA // GRPA
    ept = src_hbm.shape[0] // NS
    m = ept // CE
    cid = lax.axis_index("c")
    sid = lax.axis_index("s")
    tbase = sid * NODE_T
    boff = cid * N

    z16 = jnp.zeros((16,), jnp.float32)

    @pl.loop(0, CE)
    def _(r):
        for u in range(D // 16):
            rows0[r, pl.ds(u * 16, 16)] = z16

    for i in range(128 // 16):
        dbuf[pl.ds(i * 16, 16)] = z16

    @pl.loop(0, NODE_T // CE)
    def _(i):
        pltpu.async_copy(rows0, agg_sh.at[pl.ds(tbase + i * CE, CE)], asem)

    @pl.loop(0, NODE_T // 128)
    def _(i):
        pltpu.async_copy(dbuf, deg_sh.at[pl.ds(tbase + i * 128, 128)], asem)
        pltpu.async_copy(dbuf, cn_sh.at[pl.ds(tbase + i * 128, 128)], asem)

    @pl.loop(0, NODE_T // CE)
    def _(i):
        pltpu.make_async_copy(
            rows0, agg_sh.at[pl.ds(tbase + i * CE, CE)], asem).wait()

    @pl.loop(0, NODE_T // 128)
    def _(i):
        pltpu.make_async_copy(
            dbuf, deg_sh.at[pl.ds(tbase + i * 128, 128)], asem).wait()
        pltpu.make_async_copy(
            dbuf, cn_sh.at[pl.ds(tbase + i * 128, 128)], asem).wait()

    plsc.subcore_barrier()

    @pl.loop(0, ngA)
    def _(g):
        base = sid * nchA + g * GRPA
        pltpu.sync_copy(dstA_hbm.at[pl.ds(base, GRPA)], dst8)
        pltpu.sync_copy(ewA_hbm.at[pl.ds(base, GRPA)], ew8)

        @pl.loop(0, GRPA)
        def _(k):
            pltpu.async_copy(ew8.at[k], deg_sh.at[dst8.at[k]], asem,
                             add=True)

        @pl.loop(0, GRPA)
        def _(k):
            pltpu.make_async_copy(ew8.at[k], deg_sh.at[dst8.at[k]],
                                  asem).wait()

    plsc.subcore_barrier()

    @pl.loop(0, NODE_T // 128)
    def _(half):
        hb = tbase + half * 128
        pltpu.sync_copy(deg_sh.at[pl.ds(hb, 128)], dbuf)

        @pl.loop(0, 128 // 16)
        def _(i):
            v = dbuf[pl.ds(i * 16, 16)] + 1.0
            iv = lax.bitcast_convert_type(v, jnp.int32)
            iv = 0x5F3759DF - lax.shift_right_logical(iv, 1)
            y = lax.bitcast_convert_type(iv, jnp.float32)
            for _ in range(3):
                y = y * (1.5 - 0.5 * v * y * y)
            dbuf[pl.ds(i * 16, 16)] = y

        pltpu.sync_copy(dbuf, dinv_sh.at[pl.ds(hb, 128)])

        @pl.when(cid == 0)
        def _():
            pltpu.sync_copy(dbuf, dinv_o.at[pl.ds(hb, 128)])

    plsc.subcore_barrier()
    pltpu.sync_copy(dinv_sh.at[pl.ds(0, dinv_t.shape[0])], dinv_t)

    def prefetch(c, b):
        base = sid * ept + c * CE
        pltpu.async_copy(src_hbm.at[pl.ds(base, CE)], srcd[b], esem[b])
        pltpu.async_copy(dst_hbm.at[pl.ds(base, CE)], dstd[b], esem[b])
        pltpu.async_copy(ew_hbm.at[pl.ds(base, CE)], ewd[b], esem[b])

    def launch(c, b, rb):
        base = sid * ept + c * CE
        pltpu.make_async_copy(src_hbm.at[pl.ds(base, CE)], srcd[b],
                              esem[b]).wait()
        pltpu.make_async_copy(dst_hbm.at[pl.ds(base, CE)], dstd[b],
                              esem[b]).wait()
        pltpu.make_async_copy(ew_hbm.at[pl.ds(base, CE)], ewd[b],
                              esem[b]).wait()

        @pl.loop(0, CE // 16)
        def _(gg):
            srcv = srcd[b][pl.ds(gg * 16, 16)]
            dstv = dstd[b][pl.ds(gg * 16, 16)]
            eww = ewd[b][pl.ds(gg * 16, 16)]
            nv = plsc.load_gather(dinv_t, [srcv]) * eww * \
                plsc.load_gather(dinv_t, [dstv])
            normd[b][pl.ds(gg * 16, 16)] = nv
            srcb[b][pl.ds(gg * 16, 16)] = srcv + boff

        pltpu.async_copy(normd[b], cn_sh.at[srcd[b]], csem[b], add=True)
        pltpu.async_copy(x_hbm.at[srcb[b]], rows[rb], gsem[rb])

    def finish(c, b, rb):
        pltpu.make_async_copy(x_hbm.at[srcb[b]], rows[rb], gsem[rb]).wait()

        @pl.loop(0, CE // 16)
        def _(g):
            normv = normd[b][pl.ds(g * 16, 16)]
            for r in range(16):
                nb = jnp.take_along_axis(
                    normv, jnp.full((16, 1), r, jnp.int32)[:, 0], axis=0,
                    mode="promise_in_bounds")
                for u in range(D // 16):
                    rows[rb][g * 16 + r, pl.ds(u * 16, 16)] = \
                        rows[rb][g * 16 + r, pl.ds(u * 16, 16)] * nb

        pltpu.async_copy(rows[rb], agg_sh.at[dstd[b]], ssem[rb], add=True)

    def drain_scat(c, b, rb):
        pltpu.make_async_copy(rows[rb], agg_sh.at[dstd[b]], ssem[rb]).wait()

    def drain_cn(c, b):
        pltpu.make_async_copy(normd[b], cn_sh.at[srcd[b]], csem[b]).wait()

    prefetch(0, 0)
    prefetch(1, 1)
    prefetch(2, 2)
    launch(0, 0, 0)
    launch(1, 1, 1)

    @pl.loop(0, m // SLOTS)
    def _(t):
        y0 = t * SLOTS
        for jj in range(SLOTS):
            y = y0 + jj
            yl = y + 2
            bl, rbl = (jj + 2) % KE, (jj + 2) % KR

            @pl.when(yl < m)
            def _():
                @pl.when(yl >= KR)
                def _():
                    drain_scat(yl - KR, (yl - KR + KE) % KE if False else
                               (jj + 2 - KR) % KE, (jj + 2 - KR) % KR)

                @pl.when(yl >= KE - 2)
                def _():
                    drain_cn(yl - (KE - 2), (jj + 2 - (KE - 2)) % KE)

                launch(yl, bl, rbl)

            finish(y, jj % KE, jj % KR)

            yp = y + 3

            @pl.when(yp < m)
            def _():
                prefetch(yp, (jj + 3) % KE)

    for c in range(KR):
        cc = m - KR + c
        drain_scat(cc, cc % KE, cc % KR)
    for c in range(KE - 2):
        cc = m - (KE - 2) + c
        drain_cn(cc, cc % KE)

    plsc.subcore_barrier()

    pltpu.sync_copy(agg_sh.at[pl.ds(tbase, NODE_T)],
                    agg_o.at[pl.ds(cid * NPAD + tbase, NODE_T)])

    @pl.when(cid == 0)
    def _():
        pltpu.sync_copy(cn_sh.at[pl.ds(tbase, NODE_T)],
                        cn_o.at[pl.ds(tbase, NODE_T)])


def _make_sc_kernel(interpret=False):
    dma = pltpu.SemaphoreType.DMA
    i32 = jnp.int32
    f32 = jnp.float32
    return pl.kernel(
        _sc_body,
        out_type=(
            jax.ShapeDtypeStruct((BS * NPAD, D), f32),
            jax.ShapeDtypeStruct((NPAD,), f32),
            jax.ShapeDtypeStruct((NPAD,), f32),
        ),
        mesh=plsc.VectorSubcoreMesh(core_axis_name="c", subcore_axis_name="s",
                                    num_cores=NC),
        scratch_types=(
            [
                pltpu.VMEM_SHARED((NPAD, D), f32),
                pltpu.VMEM_SHARED((NPAD,), f32),
                pltpu.VMEM_SHARED((NPAD,), f32),
                pltpu.VMEM_SHARED((NPAD,), f32),
                pltpu.VMEM((GRPA, CEA), i32),
                pltpu.VMEM((GRPA, CEA), f32),
                pltpu.VMEM((N,), f32),
                pltpu.VMEM((128,), f32),
            ]
            + [pltpu.VMEM((CE,), i32) for _ in range(KE)]
            + [pltpu.VMEM((CE,), i32) for _ in range(KE)]
            + [pltpu.VMEM((CE,), f32) for _ in range(KE)]
            + [pltpu.VMEM((CE,), i32) for _ in range(KE)]
            + [pltpu.VMEM((CE,), f32) for _ in range(KE)]
            + [pltpu.VMEM((CE, D), f32) for _ in range(KR)]
            + [dma] * (2 * KR + 2 * KE + 1)
        ),
        compiler_params=pltpu.CompilerParams(needs_layout_passes=False),
        interpret=interpret,
    )


def _tc_body(x_ref, agg_ref, cn_ref, dinv_ref, w1_ref, b1_ref, w2_ref, b2_ref,
             o_ref):
    dv = dinv_ref[...]
    sl = dv * dv
    c = cn_ref[...] + sl
    w1 = w1_ref[...]
    w2 = w2_ref[...]
    for b in range(BS):
        t = agg_ref[b, :N, :] + sl * x_ref[b]
        h = jnp.maximum(
            jnp.dot(t, w1, preferred_element_type=jnp.float32) + b1_ref[...],
            0.0)
        p = jnp.sum(c * h, axis=0, keepdims=True)
        o_ref[b:b + 1, :] = (
            jnp.dot(p, w2, preferred_element_type=jnp.float32) * (1.0 / N)
            + b2_ref[...])


def _tc_call(x, agg, cn2, dinv2, W1, b1r, W2, b2r, interpret=False):
    return pl.pallas_call(
        _tc_body,
        out_shape=jax.ShapeDtypeStruct((BS, D), jnp.float32),
        interpret=interpret,
    )(x, agg, cn2, dinv2, W1, b1r, W2, b2r)


def _run(node_features, edge_index, edge_features, W1, b1, W2, b2,
         sc_interpret=False, tc_interpret=False):
    e = edge_index.shape[1]
    ew = edge_features[:, -1]

    nchA = (((e + NS * CEA - 1) // (NS * CEA)) + GRPA - 1) // GRPA * GRPA
    e_padA = NS * nchA * CEA
    dstA = jnp.pad(edge_index[1], (0, e_padA - e)).reshape(NS * nchA, CEA)
    ewA = jnp.pad(ew, (0, e_padA - e)).reshape(NS * nchA, CEA)

    m = ((e + NS * CE - 1) // (NS * CE) + SLOTS - 1) // SLOTS * SLOTS
    e_padC = NS * m * CE
    src1 = jnp.pad(edge_index[0], (0, e_padC - e))
    dst1 = jnp.pad(edge_index[1], (0, e_padC - e))
    ew1 = jnp.pad(ew, (0, e_padC - e))

    xflat = node_features.reshape(BS * N, D)

    sc = _make_sc_kernel(interpret=sc_interpret)
    agg_flat, cn, dinv = sc(dstA, ewA, src1, dst1, ew1, xflat)
    agg = agg_flat.reshape(BS, NPAD, D)

    cn2 = cn[:N, None]
    dinv2 = dinv[:N, None]
    return _tc_call(node_features, agg, cn2, dinv2, W1, b1[None, :], W2,
                    b2[None, :], interpret=tc_interpret)


def kernel(node_features, edge_index, edge_features, W1, b1, W2, b2):
    return _run(node_features, edge_index, edge_features, W1, b1, W2, b2)


# speedup vs baseline: 1.0714x; 1.0004x over previous
import jax
import jax.numpy as jnp
from jax import lax
from jax.experimental import pallas as pl
from jax.experimental.pallas import tpu as pltpu
from jax.experimental.pallas import tpu_sc as plsc

BS = 2
N = 10000
D = 128
NC = 2
NS = 16
CE = 80
KR = 3
KE = 6
SLOTS = 6
CEA = 128
GRPA = 8

NODE_T = ((N + NS * 16 - 1) // (NS * 16)) * 16
NPAD = NODE_T * NS


def _sc_body(dstA_hbm, ewA_hbm, src_hbm, dst_hbm, ew_hbm, x_hbm,
             agg_o, cn_o, dinv_o,
             agg_sh, deg_sh, dinv_sh, cn_sh,
             dst8, ew8, dinv_t, dbuf,
             srcd0, srcd1, srcd2, srcd3, srcd4, srcd5,
             dstd0, dstd1, dstd2, dstd3, dstd4, dstd5,
             ewd0, ewd1, ewd2, ewd3, ewd4, ewd5,
             srcb0, srcb1, srcb2, srcb3, srcb4, srcb5,
             normd0, normd1, normd2, normd3, normd4, normd5,
             rows0, rows1, rows2,
             gsem0, gsem1, gsem2, ssem0, ssem1, ssem2,
             esem0, esem1, esem2, esem3, esem4, esem5,
             csem0, csem1, csem2, csem3, csem4, csem5,
             asem):
    srcd = [srcd0, srcd1, srcd2, srcd3, srcd4, srcd5]
    dstd = [dstd0, dstd1, dstd2, dstd3, dstd4, dstd5]
    ewd = [ewd0, ewd1, ewd2, ewd3, ewd4, ewd5]
    srcb = [srcb0, srcb1, srcb2, srcb3, srcb4, srcb5]
    normd = [normd0, normd1, normd2, normd3, normd4, normd5]
    rows = [rows0, rows1, rows2]
    gsem = [gsem0, gsem1, gsem2]
    ssem = [ssem0, ssem1, ssem2]
    esem = [esem0, esem1, esem2, esem3, esem4, esem5]
    csem = [csem0, csem1, csem2, csem3, csem4, csem5]

    nchA = dstA_hbm.shape[0] // NS
    ngA = nchA // GRPA
    ept = src_hbm.shape[0] // NS
    m = ept // CE
    cid = lax.axis_index("c")
    sid = lax.axis_index("s")
    tbase = sid * NODE_T
    boff = cid * N

    z16 = jnp.zeros((16,), jnp.float32)

    @pl.loop(0, CE)
    def _(r):
        for u in range(D // 16):
            rows0[r, pl.ds(u * 16, 16)] = z16

    for i in range(128 // 16):
        dbuf[pl.ds(i * 16, 16)] = z16

    @pl.loop(0, NODE_T // CE)
    def _(i):
        pltpu.async_copy(rows0, agg_sh.at[pl.ds(tbase + i * CE, CE)], asem)

    @pl.loop(0, NODE_T // 128)
    def _(i):
        pltpu.async_copy(dbuf, deg_sh.at[pl.ds(tbase + i * 128, 128)], asem)
        pltpu.async_copy(dbuf, cn_sh.at[pl.ds(tbase + i * 128, 128)], asem)

    @pl.loop(0, NODE_T // CE)
    def _(i):
        pltpu.make_async_copy(
            rows0, agg_sh.at[pl.ds(tbase + i * CE, CE)], asem).wait()

    @pl.loop(0, NODE_T // 128)
    def _(i):
        pltpu.make_async_copy(
            dbuf, deg_sh.at[pl.ds(tbase + i * 128, 128)], asem).wait()
        pltpu.make_async_copy(
            dbuf, cn_sh.at[pl.ds(tbase + i * 128, 128)], asem).wait()

    plsc.subcore_barrier()

    @pl.loop(0, ngA)
    def _(g):
        base = sid * nchA + g * GRPA
        pltpu.sync_copy(dstA_hbm.at[pl.ds(base, GRPA)], dst8)
        pltpu.sync_copy(ewA_hbm.at[pl.ds(base, GRPA)], ew8)

        @pl.loop(0, GRPA)
        def _(k):
            pltpu.async_copy(ew8.at[k], deg_sh.at[dst8.at[k]], asem,
                             add=True)

        @pl.loop(0, GRPA)
        def _(k):
            pltpu.make_async_copy(ew8.at[k], deg_sh.at[dst8.at[k]],
                                  asem).wait()

    plsc.subcore_barrier()

    @pl.loop(0, NODE_T // 128)
    def _(half):
        hb = tbase + half * 128
        pltpu.sync_copy(deg_sh.at[pl.ds(hb, 128)], dbuf)

        @pl.loop(0, 128 // 16)
        def _(i):
            v = dbuf[pl.ds(i * 16, 16)] + 1.0
            iv = lax.bitcast_convert_type(v, jnp.int32)
            iv = 0x5F3759DF - lax.shift_right_logical(iv, 1)
            y = lax.bitcast_convert_type(iv, jnp.float32)
            for _ in range(3):
                y = y * (1.5 - 0.5 * v * y * y)
            dbuf[pl.ds(i * 16, 16)] = y

        pltpu.sync_copy(dbuf, dinv_sh.at[pl.ds(hb, 128)])

        @pl.when(cid == 0)
        def _():
            pltpu.sync_copy(dbuf, dinv_o.at[pl.ds(hb, 128)])

    plsc.subcore_barrier()
    pltpu.sync_copy(dinv_sh.at[pl.ds(0, dinv_t.shape[0])], dinv_t)

    def prefetch(c, b):
        base = sid * ept + c * CE
        pltpu.async_copy(src_hbm.at[pl.ds(base, CE)], srcd[b], esem[b])
        pltpu.async_copy(dst_hbm.at[pl.ds(base, CE)], dstd[b], esem[b])
        pltpu.async_copy(ew_hbm.at[pl.ds(base, CE)], ewd[b], esem[b])

    def launch(c, b, rb):
        base = sid * ept + c * CE
        pltpu.make_async_copy(src_hbm.at[pl.ds(base, CE)], srcd[b],
                              esem[b]).wait()
        pltpu.make_async_copy(dst_hbm.at[pl.ds(base, CE)], dstd[b],
                              esem[b]).wait()
        pltpu.make_async_copy(ew_hbm.at[pl.ds(base, CE)], ewd[b],
                              esem[b]).wait()

        @pl.loop(0, CE // 16)
        def _(gg):
            srcv = srcd[b][pl.ds(gg * 16, 16)]
            dstv = dstd[b][pl.ds(gg * 16, 16)]
            eww = ewd[b][pl.ds(gg * 16, 16)]
            nv = plsc.load_gather(dinv_t, [srcv]) * eww * \
                plsc.load_gather(dinv_t, [dstv])
            normd[b][pl.ds(gg * 16, 16)] = nv
            srcb[b][pl.ds(gg * 16, 16)] = srcv + boff

        pltpu.async_copy(normd[b], cn_sh.at[srcd[b]], csem[b], add=True)
        pltpu.async_copy(x_hbm.at[srcb[b]], rows[rb], gsem[rb])

    def finish(c, b, rb):
        pltpu.make_async_copy(x_hbm.at[srcb[b]], rows[rb], gsem[rb]).wait()

        @pl.loop(0, CE // 16)
        def _(g):
            normv = normd[b][pl.ds(g * 16, 16)]
            for r in range(16):
                nb = jnp.take_along_axis(
                    normv, jnp.full((16, 1), r, jnp.int32)[:, 0], axis=0,
                    mode="promise_in_bounds")
                for u in range(D // 16):
                    rows[rb][g * 16 + r, pl.ds(u * 16, 16)] = \
                        rows[rb][g * 16 + r, pl.ds(u * 16, 16)] * nb

        pltpu.async_copy(rows[rb], agg_sh.at[dstd[b]], ssem[rb], add=True)

    def drain_scat(c, b, rb):
        pltpu.make_async_copy(rows[rb], agg_sh.at[dstd[b]], ssem[rb]).wait()

    def drain_cn(c, b):
        pltpu.make_async_copy(normd[b], cn_sh.at[srcd[b]], csem[b]).wait()

    prefetch(0, 0)
    prefetch(1, 1)
    prefetch(2, 2)
    launch(0, 0, 0)
    launch(1, 1, 1)

    @pl.loop(0, m // SLOTS)
    def _(t):
        y0 = t * SLOTS
        for jj in range(SLOTS):
            y = y0 + jj
            yl = y + 2
            bl, rbl = (jj + 2) % KE, (jj + 2) % KR

            @pl.when(yl < m)
            def _():
                @pl.when(yl >= KR)
                def _():
                    drain_scat(yl - KR, (jj + 2 - KR) % KE,
                               (jj + 2 - KR) % KR)

                @pl.when(yl >= KE - 2)
                def _():
                    drain_cn(yl - (KE - 2), (jj + 2 - (KE - 2)) % KE)

                launch(yl, bl, rbl)

            finish(y, jj % KE, jj % KR)

            yp = y + 3

            @pl.when(yp < m)
            def _():
                prefetch(yp, (jj + 3) % KE)

    for c in range(KR):
        cc = m - KR + c
        drain_scat(cc, cc % KE, cc % KR)
    for c in range(KE - 2):
        cc = m - (KE - 2) + c
        drain_cn(cc, cc % KE)

    plsc.subcore_barrier()

    pltpu.sync_copy(agg_sh.at[pl.ds(tbase, NODE_T)],
                    agg_o.at[pl.ds(cid * NPAD + tbase, NODE_T)])

    @pl.when(cid == 0)
    def _():
        pltpu.sync_copy(cn_sh.at[pl.ds(tbase, NODE_T)],
                        cn_o.at[pl.ds(tbase, NODE_T)])


def _make_sc_kernel(interpret=False):
    dma = pltpu.SemaphoreType.DMA
    i32 = jnp.int32
    f32 = jnp.float32
    return pl.kernel(
        _sc_body,
        out_type=(
            jax.ShapeDtypeStruct((BS * NPAD, D), f32),
            jax.ShapeDtypeStruct((NPAD,), f32),
            jax.ShapeDtypeStruct((NPAD,), f32),
        ),
        mesh=plsc.VectorSubcoreMesh(core_axis_name="c", subcore_axis_name="s",
                                    num_cores=NC),
        scratch_types=(
            [
                pltpu.VMEM_SHARED((NPAD, D), f32),
                pltpu.VMEM_SHARED((NPAD,), f32),
                pltpu.VMEM_SHARED((NPAD,), f32),
                pltpu.VMEM_SHARED((NPAD,), f32),
                pltpu.VMEM((GRPA, CEA), i32),
                pltpu.VMEM((GRPA, CEA), f32),
                pltpu.VMEM((N,), f32),
                pltpu.VMEM((128,), f32),
            ]
            + [pltpu.VMEM((CE,), i32) for _ in range(KE)]
            + [pltpu.VMEM((CE,), i32) for _ in range(KE)]
            + [pltpu.VMEM((CE,), f32) for _ in range(KE)]
            + [pltpu.VMEM((CE,), i32) for _ in range(KE)]
            + [pltpu.VMEM((CE,), f32) for _ in range(KE)]
            + [pltpu.VMEM((CE, D), f32) for _ in range(KR)]
            + [dma] * (2 * KR + 2 * KE + 1)
        ),
        compiler_params=pltpu.CompilerParams(needs_layout_passes=False),
        interpret=interpret,
    )


def _tc_body(x_ref, agg_ref, cn_ref, dinv_ref, w1_ref, b1_ref, w2_ref, b2_ref,
             o_ref):
    dv = dinv_ref[...]
    sl = dv * dv
    c = cn_ref[...] + sl
    w1 = w1_ref[...]
    w2 = w2_ref[...]
    for b in range(BS):
        t = agg_ref[b, :N, :] + sl * x_ref[b]
        h = jnp.maximum(
            jnp.dot(t, w1, preferred_element_type=jnp.float32) + b1_ref[...],
            0.0)
        p = jnp.sum(c * h, axis=0, keepdims=True)
        o_ref[b:b + 1, :] = (
            jnp.dot(p, w2, preferred_element_type=jnp.float32) * (1.0 / N)
            + b2_ref[...])


def _tc_call(x, agg, cn2, dinv2, W1, b1r, W2, b2r, interpret=False):
    return pl.pallas_call(
        _tc_body,
        out_shape=jax.ShapeDtypeStruct((BS, D), jnp.float32),
        interpret=interpret,
    )(x, agg, cn2, dinv2, W1, b1r, W2, b2r)


def _run(node_features, edge_index, edge_features, W1, b1, W2, b2,
         sc_interpret=False, tc_interpret=False):
    e = edge_index.shape[1]
    ew = edge_features[:, -1]

    nchA = (((e + NS * CEA - 1) // (NS * CEA)) + GRPA - 1) // GRPA * GRPA
    e_padA = NS * nchA * CEA
    dstA = jnp.pad(edge_index[1], (0, e_padA - e)).reshape(NS * nchA, CEA)
    ewA = jnp.pad(ew, (0, e_padA - e)).reshape(NS * nchA, CEA)

    m = ((e + NS * CE - 1) // (NS * CE) + SLOTS - 1) // SLOTS * SLOTS
    e_padC = NS * m * CE
    src1 = jnp.pad(edge_index[0], (0, e_padC - e))
    dst1 = jnp.pad(edge_index[1], (0, e_padC - e))
    ew1 = jnp.pad(ew, (0, e_padC - e))

    xflat = node_features.reshape(BS * N, D)

    sc = _make_sc_kernel(interpret=sc_interpret)
    agg_flat, cn, dinv = sc(dstA, ewA, src1, dst1, ew1, xflat)
    agg = agg_flat.reshape(BS, NPAD, D)

    cn2 = cn[:N, None]
    dinv2 = dinv[:N, None]
    return _tc_call(node_features, agg, cn2, dinv2, W1, b1[None, :], W2,
                    b2[None, :], interpret=tc_interpret)


def kernel(node_features, edge_index, edge_features, W1, b1, W2, b2):
    return _run(node_features, edge_index, edge_features, W1, b1, W2, b2)
